# baseline (speedup 1.0000x reference)
import functools

import jax
import jax.numpy as jnp
from jax import lax
from jax.experimental import pallas as pl
from jax.experimental.pallas import tpu as pltpu
from jax.experimental.pallas import tpu_sc as plsc

N = 10000
E = 320000
D_IN = 128
HID = 128
HID2 = 256
DPAD = 16

NC = 2
NS = 16
NW = NC * NS
EPT = E // NW
CW = 40
CN = 1000
RPT = 624
TAIL = N - NS * RPT

B = 2000
B3 = 5000

_mesh = plsc.VectorSubcoreMesh(
    core_axis_name="c", subcore_axis_name="s", num_cores=NC, num_subcores=NS)


def _zero_acc(zeros_hbm, acc, s):
    pltpu.sync_copy(zeros_hbm.at[pl.ds(s * RPT, RPT)], acc.at[pl.ds(s * RPT, RPT)])

    @pl.when(s == NS - 1)
    def _():
        pltpu.sync_copy(zeros_hbm.at[pl.ds(NS * RPT, TAIL)],
                        acc.at[pl.ds(NS * RPT, TAIL)])


def _write_out(acc, out_hbm, c, s):
    pltpu.sync_copy(acc.at[pl.ds(s * RPT, RPT)], out_hbm.at[c, pl.ds(s * RPT, RPT)])

    @pl.when(s == NS - 1)
    def _():
        pltpu.sync_copy(acc.at[pl.ds(NS * RPT, TAIL)],
                        out_hbm.at[c, pl.ds(NS * RPT, TAIL)])


def _deg_body(ei, ones_hbm, zeros_hbm, out_hbm, colbuf, onesbuf, acc,
              ss0, ss1, ss2):
    nch = EPT // CN
    ss = (ss0, ss1, ss2)
    c = lax.axis_index("c")
    s = lax.axis_index("s")
    wid = s * NC + c
    _zero_acc(zeros_hbm, acc, s)
    pltpu.sync_copy(ei.at[1, pl.ds(wid * EPT, EPT)], colbuf)
    pltpu.sync_copy(ones_hbm, onesbuf)
    plsc.subcore_barrier()

    def cidx(k):
        return colbuf.at[pl.ds(k * CN, CN)]

    for k in range(nch):
        j = k % 3
        if k >= 3:
            pltpu.make_async_copy(onesbuf, acc.at[cidx(k - 3)], ss[j]).wait()
        pltpu.async_copy(onesbuf, acc.at[cidx(k)], ss[j], add=True)
    for k in range(nch - 3, nch):
        j = k % 3
        pltpu.make_async_copy(onesbuf, acc.at[cidx(k)], ss[j]).wait()
    plsc.subcore_barrier()
    _write_out(acc, out_hbm, c, s)


_deg_call = pl.kernel(
    _deg_body,
    out_type=jax.ShapeDtypeStruct((NC, N, DPAD), jnp.float32),
    mesh=_mesh,
    compiler_params=pltpu.CompilerParams(use_tc_tiling_on_sc=False),
    scratch_types=[
        pltpu.VMEM((EPT,), jnp.int32),
        pltpu.VMEM((CN, DPAD), jnp.float32),
        pltpu.VMEM_SHARED((N, DPAD), jnp.float32),
        pltpu.SemaphoreType.DMA,
        pltpu.SemaphoreType.DMA,
        pltpu.SemaphoreType.DMA,
    ],
)


NSLOT = 6
LEAD = 3


def _make_agg_body(nch, chunk):
    def body(hs, ei, zeros_hbm, out_hbm, *scr):
        rowbuf, colbuf = scr[0], scr[1]
        gb = scr[2:2 + NSLOT]
        acc = scr[2 + NSLOT]
        gs = scr[3 + NSLOT:3 + 2 * NSLOT]
        ss = scr[3 + 2 * NSLOT:3 + 3 * NSLOT]
        c = lax.axis_index("c")
        s = lax.axis_index("s")
        wid = s * NC + c
        _zero_acc(zeros_hbm, acc, s)
        pltpu.sync_copy(ei.at[0, pl.ds(wid * EPT, EPT)], rowbuf)
        pltpu.sync_copy(ei.at[1, pl.ds(wid * EPT, EPT)], colbuf)
        plsc.subcore_barrier()

        def ridx(k):
            return rowbuf.at[pl.ds(k * chunk, chunk)]

        def cidx(k):
            return colbuf.at[pl.ds(k * chunk, chunk)]

        def step(k, j, jr, do_refill, refill_waits):
            if do_refill:
                if refill_waits:
                    pltpu.make_async_copy(
                        gb[jr], acc.at[cidx(k)], ss[jr]).wait()
                pltpu.async_copy(hs.at[ridx(k + LEAD)], gb[jr], gs[jr])
            pltpu.make_async_copy(hs.at[ridx(k)], gb[j], gs[j]).wait()
            pltpu.async_copy(gb[j], acc.at[cidx(k)], ss[j], add=True)

        for j in range(min(LEAD, nch)):
            pltpu.async_copy(hs.at[ridx(j)], gb[j], gs[j])

        head_end = min(NSLOT, nch)
        for k in range(head_end):
            step(k, k % NSLOT, (k + LEAD) % NSLOT,
                 k + LEAD < nch, k + LEAD >= NSLOT)
        main_end = max(head_end, nch - LEAD - 1)
        n_rounds = (main_end - head_end) // NSLOT
        main_end = head_end + n_rounds * NSLOT

        def round_(i, carry):
            base = head_end + i * NSLOT
            for j in range(NSLOT):
                k = base + j
                step(k, (head_end + j) % NSLOT,
                     (head_end + j + LEAD) % NSLOT, True, True)
            return carry

        if n_rounds:
            lax.fori_loop(0, n_rounds, round_, 0)
        for k in range(main_end, nch):
            step(k, k % NSLOT, (k + LEAD) % NSLOT,
                 k + LEAD < nch, k + LEAD >= NSLOT)
        for k in range(max(0, nch - NSLOT), nch):
            j = k % NSLOT
            pltpu.make_async_copy(gb[j], acc.at[cidx(k)], ss[j]).wait()
        plsc.subcore_barrier()
        _write_out(acc, out_hbm, c, s)

    return body


def _make_agg(d, chunk):
    nch = EPT // chunk
    return pl.kernel(
        _make_agg_body(nch, chunk),
        out_type=jax.ShapeDtypeStruct((NC, N, d), jnp.float32),
        mesh=_mesh,
        compiler_params=pltpu.CompilerParams(use_tc_tiling_on_sc=False),
        scratch_types=(
            [pltpu.VMEM((EPT,), jnp.int32), pltpu.VMEM((EPT,), jnp.int32)]
            + [pltpu.VMEM((chunk, d), jnp.float32)] * NSLOT
            + [pltpu.VMEM_SHARED((N, d), jnp.float32)]
            + [pltpu.SemaphoreType.DMA] * (2 * NSLOT)
        ),
    )


_agg_wide = _make_agg(HID, CW)
_agg_narrow = _make_agg(DPAD, CN)


def _k0_body(x_ref, dp_ref, hs_ref, dinv_ref):
    deg = 1.0 + dp_ref[0, :, 0:1] + dp_ref[1, :, 0:1]
    dinv = lax.rsqrt(deg)
    hs_ref[...] = x_ref[...] * dinv
    dinv_ref[...] = jnp.broadcast_to(dinv, dinv_ref.shape)


def _k1_body(p_ref, hs_ref, dinv_ref, w_ref, b_ref, out_ref):
    dinv = dinv_ref[:, 0:1]
    g = dinv * (p_ref[0] + p_ref[1] + hs_ref[...])
    h = jnp.dot(g, w_ref[...], preferred_element_type=jnp.float32) + b_ref[...]
    out_ref[...] = dinv * jnp.maximum(h, 0.0)


def _k2_body(p_ref, hs_ref, dinv_ref, w2_ref, b2_ref, w3_ref, out_ref):
    dinv = dinv_ref[:, 0:1]
    g = dinv * (p_ref[0] + p_ref[1] + hs_ref[...])
    z = jnp.dot(g, w2_ref[...], preferred_element_type=jnp.float32) + b2_ref[...]
    z = jnp.maximum(z, 0.0)
    out_ref[...] = dinv * jnp.dot(z, w3_ref[...], preferred_element_type=jnp.float32)


def _k3_body(p_ref, hs_ref, dinv_ref, b3_ref, out_ref):
    dinv = dinv_ref[:, 0:1]
    g = dinv * (p_ref[0] + p_ref[1] + hs_ref[...]) + b3_ref[...]
    mask = lax.broadcasted_iota(jnp.int32, g.shape, 1) < 2
    m = jnp.max(jnp.where(mask, g, -jnp.inf), axis=1, keepdims=True)
    e = jnp.where(mask, jnp.exp(g - m), 0.0)
    r = g - (m + jnp.log(jnp.sum(e, axis=1, keepdims=True)))
    out_ref[...] = r[:, 0:2]


def _row_spec(d):
    return pl.BlockSpec((B, d), lambda i: (i, 0))


def _p_spec(d):
    return pl.BlockSpec((NC, B, d), lambda i: (0, i, 0))


def _full_spec(*shape):
    return pl.BlockSpec(shape, lambda i: (0,) * len(shape))


_k0_call = pl.pallas_call(
    _k0_body,
    grid=(N // B,),
    in_specs=[_row_spec(D_IN), _p_spec(DPAD)],
    out_specs=[_row_spec(D_IN), _row_spec(DPAD)],
    out_shape=[jax.ShapeDtypeStruct((N, D_IN), jnp.float32),
               jax.ShapeDtypeStruct((N, DPAD), jnp.float32)],
)

_k1_call = pl.pallas_call(
    _k1_body,
    grid=(N // B,),
    in_specs=[_p_spec(HID), _row_spec(HID), _row_spec(DPAD),
              _full_spec(D_IN, HID), _full_spec(1, HID)],
    out_specs=_row_spec(HID),
    out_shape=jax.ShapeDtypeStruct((N, HID), jnp.float32),
)

_k2_call = pl.pallas_call(
    _k2_body,
    grid=(N // B,),
    in_specs=[_p_spec(HID), _row_spec(HID), _row_spec(DPAD),
              _full_spec(HID, HID2), _full_spec(1, HID2), _full_spec(HID2, DPAD)],
    out_specs=_row_spec(DPAD),
    out_shape=jax.ShapeDtypeStruct((N, DPAD), jnp.float32),
)

_k3_call = pl.pallas_call(
    _k3_body,
    grid=(N // B3,),
    in_specs=[pl.BlockSpec((NC, B3, DPAD), lambda i: (0, i, 0)),
              pl.BlockSpec((B3, DPAD), lambda i: (i, 0)),
              pl.BlockSpec((B3, DPAD), lambda i: (i, 0)),
              _full_spec(1, DPAD)],
    out_specs=pl.BlockSpec((B3, 2), lambda i: (i, 0)),
    out_shape=jax.ShapeDtypeStruct((N, 2), jnp.float32),
)


def kernel(x, edge_index, W1, b1, W2, b2, W3, b3):
    ei = edge_index.astype(jnp.int32)
    zeros_w = jnp.zeros((N, HID), jnp.float32)
    zeros_n = jnp.zeros((N, DPAD), jnp.float32)
    ones_c = jnp.ones((CN, DPAD), jnp.float32)

    dp = _deg_call(ei, ones_c, zeros_n)
    hs1, dinv = _k0_call(x, dp)
    p1 = _agg_wide(hs1, ei, zeros_w)
    hs2 = _k1_call(p1, hs1, dinv, W1, b1.reshape(1, HID))
    p2 = _agg_wide(hs2, ei, zeros_w)
    w3p = jnp.pad(W3, ((0, 0), (0, DPAD - W3.shape[1])))
    hs3 = _k2_call(p2, hs2, dinv, W2, b2.reshape(1, HID2), w3p)
    p3 = _agg_narrow(hs3, ei, zeros_n)
    b3p = jnp.pad(b3, (0, DPAD - b3.shape[0])).reshape(1, DPAD)
    return _k3_call(p3, hs3, dinv, b3p)


# speedup vs baseline: 1.0245x; 1.0245x over previous
import functools

import jax
import jax.numpy as jnp
from jax import lax
from jax.experimental import pallas as pl
from jax.experimental.pallas import tpu as pltpu
from jax.experimental.pallas import tpu_sc as plsc

N = 10000
E = 320000
D_IN = 128
HID = 128
HID2 = 256
DPAD = 16

NC = 2
NS = 16
NW = NC * NS
EPT = E // NW
CW = 40
CN = 1000
RPT = 624
TAIL = N - NS * RPT

B = 2000
B3 = 5000

_mesh = plsc.VectorSubcoreMesh(
    core_axis_name="c", subcore_axis_name="s", num_cores=NC, num_subcores=NS)


def _zero_acc_start(zeros_hbm, acc, s, zsem):
    pltpu.async_copy(zeros_hbm.at[pl.ds(s * RPT, RPT)],
                     acc.at[pl.ds(s * RPT, RPT)], zsem)

    @pl.when(s == NS - 1)
    def _():
        pltpu.async_copy(zeros_hbm.at[pl.ds(NS * RPT, TAIL)],
                         acc.at[pl.ds(NS * RPT, TAIL)], zsem)


def _zero_acc_wait(zeros_hbm, acc, s, zsem):
    pltpu.make_async_copy(zeros_hbm.at[pl.ds(s * RPT, RPT)],
                          acc.at[pl.ds(s * RPT, RPT)], zsem).wait()

    @pl.when(s == NS - 1)
    def _():
        pltpu.make_async_copy(zeros_hbm.at[pl.ds(NS * RPT, TAIL)],
                              acc.at[pl.ds(NS * RPT, TAIL)], zsem).wait()


def _write_out(acc, out_hbm, c, s):
    pltpu.sync_copy(acc.at[pl.ds(s * RPT, RPT)], out_hbm.at[c, pl.ds(s * RPT, RPT)])

    @pl.when(s == NS - 1)
    def _():
        pltpu.sync_copy(acc.at[pl.ds(NS * RPT, TAIL)],
                        out_hbm.at[c, pl.ds(NS * RPT, TAIL)])


def _deg_body(ei, ones_hbm, zeros_hbm, out_hbm, colbuf, onesbuf, acc,
              ss0, ss1, ss2, zsem):
    nch = EPT // CN
    ss = (ss0, ss1, ss2)
    c = lax.axis_index("c")
    s = lax.axis_index("s")
    wid = s * NC + c
    _zero_acc_start(zeros_hbm, acc, s, zsem)
    pltpu.sync_copy(ei.at[1, pl.ds(wid * EPT, EPT)], colbuf)
    pltpu.sync_copy(ones_hbm, onesbuf)
    _zero_acc_wait(zeros_hbm, acc, s, zsem)
    plsc.subcore_barrier()

    def cidx(k):
        return colbuf.at[pl.ds(k * CN, CN)]

    for k in range(nch):
        j = k % 3
        if k >= 3:
            pltpu.make_async_copy(onesbuf, acc.at[cidx(k - 3)], ss[j]).wait()
        pltpu.async_copy(onesbuf, acc.at[cidx(k)], ss[j], add=True)
    for k in range(nch - 3, nch):
        j = k % 3
        pltpu.make_async_copy(onesbuf, acc.at[cidx(k)], ss[j]).wait()
    plsc.subcore_barrier()
    _write_out(acc, out_hbm, c, s)


_deg_call = pl.kernel(
    _deg_body,
    out_type=jax.ShapeDtypeStruct((NC, N, DPAD), jnp.float32),
    mesh=_mesh,
    compiler_params=pltpu.CompilerParams(use_tc_tiling_on_sc=False),
    scratch_types=[
        pltpu.VMEM((EPT,), jnp.int32),
        pltpu.VMEM((CN, DPAD), jnp.float32),
        pltpu.VMEM_SHARED((N, DPAD), jnp.float32),
        pltpu.SemaphoreType.DMA,
        pltpu.SemaphoreType.DMA,
        pltpu.SemaphoreType.DMA,
        pltpu.SemaphoreType.DMA,
    ],
)


NSLOT = 6
LEAD = 3


def _make_agg_body(nch, chunk):
    def body(hs, ei, zeros_hbm, out_hbm, *scr):
        rowbuf, colbuf = scr[0], scr[1]
        gb = scr[2:2 + NSLOT]
        acc = scr[2 + NSLOT]
        gs = scr[3 + NSLOT:3 + 2 * NSLOT]
        ss = scr[3 + 2 * NSLOT:3 + 3 * NSLOT]
        zsem = scr[3 + 3 * NSLOT]
        c = lax.axis_index("c")
        s = lax.axis_index("s")
        wid = s * NC + c
        _zero_acc_start(zeros_hbm, acc, s, zsem)
        pltpu.sync_copy(ei.at[0, pl.ds(wid * EPT, EPT)], rowbuf)
        pltpu.sync_copy(ei.at[1, pl.ds(wid * EPT, EPT)], colbuf)

        def ridx(k):
            return rowbuf.at[pl.ds(k * chunk, chunk)]

        def cidx(k):
            return colbuf.at[pl.ds(k * chunk, chunk)]

        def step(k, j, jr, do_refill, refill_waits):
            if do_refill:
                if refill_waits:
                    pltpu.make_async_copy(
                        gb[jr], acc.at[cidx(k)], ss[jr]).wait()
                pltpu.async_copy(hs.at[ridx(k + LEAD)], gb[jr], gs[jr])
            pltpu.make_async_copy(hs.at[ridx(k)], gb[j], gs[j]).wait()
            pltpu.async_copy(gb[j], acc.at[cidx(k)], ss[j], add=True)

        for j in range(min(LEAD, nch)):
            pltpu.async_copy(hs.at[ridx(j)], gb[j], gs[j])
        _zero_acc_wait(zeros_hbm, acc, s, zsem)
        plsc.subcore_barrier()

        head_end = min(NSLOT, nch)
        for k in range(head_end):
            step(k, k % NSLOT, (k + LEAD) % NSLOT,
                 k + LEAD < nch, k + LEAD >= NSLOT)
        main_end = max(head_end, nch - LEAD - 1)
        n_rounds = (main_end - head_end) // NSLOT
        main_end = head_end + n_rounds * NSLOT

        def round_(i, carry):
            base = head_end + i * NSLOT
            for j in range(NSLOT):
                k = base + j
                step(k, (head_end + j) % NSLOT,
                     (head_end + j + LEAD) % NSLOT, True, True)
            return carry

        if n_rounds:
            lax.fori_loop(0, n_rounds, round_, 0)
        for k in range(main_end, nch):
            step(k, k % NSLOT, (k + LEAD) % NSLOT,
                 k + LEAD < nch, k + LEAD >= NSLOT)
        for k in range(max(0, nch - NSLOT), nch):
            j = k % NSLOT
            pltpu.make_async_copy(gb[j], acc.at[cidx(k)], ss[j]).wait()
        plsc.subcore_barrier()
        _write_out(acc, out_hbm, c, s)

    return body


def _make_agg(d, chunk):
    nch = EPT // chunk
    return pl.kernel(
        _make_agg_body(nch, chunk),
        out_type=jax.ShapeDtypeStruct((NC, N, d), jnp.float32),
        mesh=_mesh,
        compiler_params=pltpu.CompilerParams(use_tc_tiling_on_sc=False),
        scratch_types=(
            [pltpu.VMEM((EPT,), jnp.int32), pltpu.VMEM((EPT,), jnp.int32)]
            + [pltpu.VMEM((chunk, d), jnp.float32)] * NSLOT
            + [pltpu.VMEM_SHARED((N, d), jnp.float32)]
            + [pltpu.SemaphoreType.DMA] * (2 * NSLOT + 1)
        ),
    )


_agg_wide = _make_agg(HID, CW)
_agg_narrow = _make_agg(DPAD, CN)


def _k0_body(x_ref, dp_ref, hs_ref, dinv_ref):
    deg = 1.0 + dp_ref[0, :, 0:1] + dp_ref[1, :, 0:1]
    dinv = lax.rsqrt(deg)
    hs_ref[...] = x_ref[...] * dinv
    dinv_ref[...] = jnp.broadcast_to(dinv, dinv_ref.shape)


def _k1_body(p_ref, hs_ref, dinv_ref, w_ref, b_ref, out_ref):
    dinv = dinv_ref[:, 0:1]
    g = dinv * (p_ref[0] + p_ref[1] + hs_ref[...])
    h = jnp.dot(g, w_ref[...], preferred_element_type=jnp.float32) + b_ref[...]
    out_ref[...] = dinv * jnp.maximum(h, 0.0)


def _k2_body(p_ref, hs_ref, dinv_ref, w2_ref, b2_ref, w3_ref, out_ref):
    dinv = dinv_ref[:, 0:1]
    g = dinv * (p_ref[0] + p_ref[1] + hs_ref[...])
    z = jnp.dot(g, w2_ref[...], preferred_element_type=jnp.float32) + b2_ref[...]
    z = jnp.maximum(z, 0.0)
    out_ref[...] = dinv * jnp.dot(z, w3_ref[...], preferred_element_type=jnp.float32)


def _k3_body(p_ref, hs_ref, dinv_ref, b3_ref, out_ref):
    dinv = dinv_ref[:, 0:1]
    g = dinv * (p_ref[0] + p_ref[1] + hs_ref[...]) + b3_ref[...]
    mask = lax.broadcasted_iota(jnp.int32, g.shape, 1) < 2
    m = jnp.max(jnp.where(mask, g, -jnp.inf), axis=1, keepdims=True)
    e = jnp.where(mask, jnp.exp(g - m), 0.0)
    r = g - (m + jnp.log(jnp.sum(e, axis=1, keepdims=True)))
    out_ref[...] = r[:, 0:2]


def _row_spec(d):
    return pl.BlockSpec((B, d), lambda i: (i, 0))


def _p_spec(d):
    return pl.BlockSpec((NC, B, d), lambda i: (0, i, 0))


def _full_spec(*shape):
    return pl.BlockSpec(shape, lambda i: (0,) * len(shape))


_k0_call = pl.pallas_call(
    _k0_body,
    grid=(N // B,),
    in_specs=[_row_spec(D_IN), _p_spec(DPAD)],
    out_specs=[_row_spec(D_IN), _row_spec(DPAD)],
    out_shape=[jax.ShapeDtypeStruct((N, D_IN), jnp.float32),
               jax.ShapeDtypeStruct((N, DPAD), jnp.float32)],
)

_k1_call = pl.pallas_call(
    _k1_body,
    grid=(N // B,),
    in_specs=[_p_spec(HID), _row_spec(HID), _row_spec(DPAD),
              _full_spec(D_IN, HID), _full_spec(1, HID)],
    out_specs=_row_spec(HID),
    out_shape=jax.ShapeDtypeStruct((N, HID), jnp.float32),
)

_k2_call = pl.pallas_call(
    _k2_body,
    grid=(N // B,),
    in_specs=[_p_spec(HID), _row_spec(HID), _row_spec(DPAD),
              _full_spec(HID, HID2), _full_spec(1, HID2), _full_spec(HID2, DPAD)],
    out_specs=_row_spec(DPAD),
    out_shape=jax.ShapeDtypeStruct((N, DPAD), jnp.float32),
)

_k3_call = pl.pallas_call(
    _k3_body,
    grid=(N // B3,),
    in_specs=[pl.BlockSpec((NC, B3, DPAD), lambda i: (0, i, 0)),
              pl.BlockSpec((B3, DPAD), lambda i: (i, 0)),
              pl.BlockSpec((B3, DPAD), lambda i: (i, 0)),
              _full_spec(1, DPAD)],
    out_specs=pl.BlockSpec((B3, 2), lambda i: (i, 0)),
    out_shape=jax.ShapeDtypeStruct((N, 2), jnp.float32),
)


def kernel(x, edge_index, W1, b1, W2, b2, W3, b3):
    ei = edge_index.astype(jnp.int32)
    zeros_w = jnp.zeros((N, HID), jnp.float32)
    zeros_n = jnp.zeros((N, DPAD), jnp.float32)
    ones_c = jnp.ones((CN, DPAD), jnp.float32)

    dp = _deg_call(ei, ones_c, zeros_n)
    hs1, dinv = _k0_call(x, dp)
    p1 = _agg_wide(hs1, ei, zeros_w)
    hs2 = _k1_call(p1, hs1, dinv, W1, b1.reshape(1, HID))
    p2 = _agg_wide(hs2, ei, zeros_w)
    w3p = jnp.pad(W3, ((0, 0), (0, DPAD - W3.shape[1])))
    hs3 = _k2_call(p2, hs2, dinv, W2, b2.reshape(1, HID2), w3p)
    p3 = _agg_narrow(hs3, ei, zeros_n)
    b3p = jnp.pad(b3, (0, DPAD - b3.shape[0])).reshape(1, DPAD)
    return _k3_call(p3, hs3, dinv, b3p)
